# baseline (speedup 1.0000x reference)
import functools

import numpy as np
import jax
import jax.numpy as jnp
from jax.experimental import pallas as pl
from jax.experimental.pallas import tpu as pltpu

_S = 128


def _upsample_matrix(n_in, n_out):
    pos = np.arange(n_out) * (n_in - 1) / (n_out - 1)
    lo = np.clip(np.floor(pos).astype(np.int64), 0, n_in - 2)
    frac = (pos - lo).astype(np.float32)
    m = np.zeros((n_out, n_in), np.float32)
    m[np.arange(n_out), lo] += 1.0 - frac
    m[np.arange(n_out), lo + 1] += frac
    return m


def _block_kernel(x_ref, m_ref, w1_ref, b1_ref, w2_ref, bout_ref, wsc_ref,
                  o_ref, pb1_ref, p1_ref, pb2_ref, p2_ref, *, H, W, C, Ho, Wo, Co):
    x = x_ref[...]
    xcat = jnp.concatenate([jnp.maximum(x, 0.0), x], axis=2)
    m = m_ref[...]
    t = jnp.dot(m, xcat.reshape(H, W * 2 * C),
                preferred_element_type=jnp.float32)
    tt = jnp.transpose(t.reshape(Ho, W, 2 * C), (1, 0, 2))
    u = jnp.dot(m, tt.reshape(W, Ho * 2 * C),
                preferred_element_type=jnp.float32)
    u = u.reshape(Wo, Ho, 2 * C)

    pb1_ref[...] = jnp.zeros_like(pb1_ref)
    pb1_ref[1:1 + Wo, 1:1 + Ho, :] = u[:, :, :C].astype(jnp.bfloat16)
    f1 = pb1_ref[...].reshape((Wo + 3) * _S, C)
    np1 = (Wo + 2) * _S
    for kc in range(3):
        p1_ref[:, kc * C:(kc + 1) * C] = f1[kc:kc + np1, :]
    mo = Wo * _S
    acc = jnp.dot(p1_ref[0:mo], w1_ref[0], preferred_element_type=jnp.float32)
    acc = acc + jnp.dot(p1_ref[_S:_S + mo], w1_ref[1],
                        preferred_element_type=jnp.float32)
    acc = acc + jnp.dot(p1_ref[2 * _S:2 * _S + mo], w1_ref[2],
                        preferred_element_type=jnp.float32)
    h = jnp.maximum(acc + b1_ref[...], 0.0).astype(jnp.bfloat16)

    pb2_ref[...] = jnp.zeros_like(pb2_ref)
    pb2_ref[1:1 + Wo, 1:1 + Ho, :] = h.reshape(Wo, _S, Co)[:, :Ho, :]
    f2 = pb2_ref[...].reshape((Wo + 3) * _S, Co)
    for kc in range(3):
        p2_ref[:, kc * Co:(kc + 1) * Co] = f2[kc:kc + np1, :]
    acc2 = jnp.dot(p2_ref[0:mo], w2_ref[0], preferred_element_type=jnp.float32)
    acc2 = acc2 + jnp.dot(p2_ref[_S:_S + mo], w2_ref[1],
                          preferred_element_type=jnp.float32)
    acc2 = acc2 + jnp.dot(p2_ref[2 * _S:2 * _S + mo], w2_ref[2],
                          preferred_element_type=jnp.float32)

    us = u[:, :, C:].astype(jnp.bfloat16).reshape(Wo * Ho, C)
    sc = jnp.dot(us, wsc_ref[...], preferred_element_type=jnp.float32)
    out = (acc2.reshape(Wo, _S, Co)[:, :Ho, :]
           + sc.reshape(Wo, Ho, Co) + bout_ref[...])
    o_ref[...] = out


def kernel(w1, b1, w2, b2, wsc, bsc, x):
    B, Cin, H, W = x.shape
    Co = w1.shape[-1]
    Ho, Wo = 2 * H, 2 * W
    xh = jnp.transpose(x, (0, 2, 3, 1))
    m = jnp.asarray(_upsample_matrix(H, Ho))
    w1t = jnp.transpose(w1[:, :, :Cin, :], (1, 0, 2, 3)) \
             .reshape(3, 3 * Cin, Co).astype(jnp.bfloat16)
    w2t = jnp.transpose(w2, (1, 0, 2, 3)) \
             .reshape(3, 3 * Co, Co).astype(jnp.bfloat16)
    wsct = wsc[:Cin, :].astype(jnp.bfloat16)
    bout = (b2 + bsc).reshape(1, Co)

    kern = functools.partial(_block_kernel, H=H, W=W, C=Cin, Ho=Ho, Wo=Wo, Co=Co)
    out_t = pl.pallas_call(
        kern,
        out_shape=jax.ShapeDtypeStruct((B, Wo, Ho, Co), jnp.float32),
        grid_spec=pltpu.PrefetchScalarGridSpec(
            num_scalar_prefetch=0,
            grid=(B,),
            in_specs=[
                pl.BlockSpec((None, H, W, Cin), lambda b: (b, 0, 0, 0)),
                pl.BlockSpec((Ho, H), lambda b: (0, 0)),
                pl.BlockSpec((3, 3 * Cin, Co), lambda b: (0, 0, 0)),
                pl.BlockSpec((1, Co), lambda b: (0, 0)),
                pl.BlockSpec((3, 3 * Co, Co), lambda b: (0, 0, 0)),
                pl.BlockSpec((1, Co), lambda b: (0, 0)),
                pl.BlockSpec((Cin, Co), lambda b: (0, 0)),
            ],
            out_specs=pl.BlockSpec((None, Wo, Ho, Co), lambda b: (b, 0, 0, 0)),
            scratch_shapes=[
                pltpu.VMEM(((Wo + 3), _S, Cin), jnp.bfloat16),
                pltpu.VMEM(((Wo + 2) * _S, 3 * Cin), jnp.bfloat16),
                pltpu.VMEM(((Wo + 3), _S, Co), jnp.bfloat16),
                pltpu.VMEM(((Wo + 2) * _S, 3 * Co), jnp.bfloat16),
            ],
        ),
        compiler_params=pltpu.CompilerParams(dimension_semantics=("parallel",)),
    )(xh, m, w1t, b1.reshape(1, Co), w2t, bout, wsct)
    return jnp.transpose(out_t, (0, 3, 2, 1))
